# Initial kernel scaffold; baseline (speedup 1.0000x reference)
import jax
import jax.numpy as jnp
from jax.experimental import pallas as pl


def kernel(points, features, frames, W1, g1, b1, W2, g2, b2, W3, g3, b3, Wsc, gsc, bsc):
    raise NotImplementedError("write your pallas kernel here")


# speedup vs baseline: 84.3783x; 84.3783x over previous
import functools

import numpy as np
import jax
import jax.numpy as jnp
from jax import lax
from jax.experimental import pallas as pl
from jax.experimental.pallas import tpu as pltpu
from jax.experimental.pallas import tpu_sc as plsc

KNB = 16
EPS = 1e-5
B, N, DIN = 4, 4096, 16
NN = B * N
E = NN * KNB
KP1 = KNB + 1
CH = 16
NC = N // CH

RA = 256
RC = 256
ND = 512
RE = 16384

_F32 = jnp.float32

def _sel_mats():
    G = np.zeros((4, 16, 16), np.float32)
    Q = np.zeros((4, 16, 16), np.float32)
    H = np.zeros((4, 16, 16), np.float32)
    for b in range(4):
        for v in range(4):
            for a in range(4):
                G[b, v * 4 + b, v * 4 + a] = 1.0
                Q[b, a * 4 + b, v * 4 + a] = 1.0
                H[b, b * 4 + a, v * 4 + a] = 1.0
    return G, Q, H

_GM, _QM, _HM = _sel_mats()
_EREP = np.zeros((32, KP1 * CH), np.float32)
for _t in range(KP1):
    _EREP[_t, _t * CH:(_t + 1) * CH] = 1.0
_SELM = np.concatenate([_HM.reshape(64, 16), _GM.reshape(64, 16)], axis=0)
_L1M = np.concatenate([_QM.reshape(64, 16), _GM.reshape(64, 16)], axis=0)


def _mm(a, b):
    return lax.dot_general(a, b, (((1,), (0,)), ((), ())),
                           preferred_element_type=_F32)


def _knn_body(pts_ref, prow_ref, score_ref, gidx_ref):
    p = pts_ref[0]
    rid = pl.program_id(1)
    prow = prow_ref[0]
    xx = jnp.sum(p * p, axis=0, keepdims=True)
    d = lax.dot_general(prow, p, (((0,), (0,)), ((), ())),
                        preferred_element_type=_F32)
    score = 2.0 * d - xx
    score_ref[...] = score
    m = jnp.max(score.reshape(RA, NC, CH), axis=2)
    ci = lax.broadcasted_iota(jnp.int32, (RA, NC), 1)
    node = lax.broadcasted_iota(jnp.int32, (RA, 1), 0)
    base = (pl.program_id(0) * N + rid * RA) * NC
    neg = _F32(-jnp.inf)
    outs = []
    for _ in range(KP1):
        gm = jnp.max(m, axis=1, keepdims=True)
        sel = jnp.min(jnp.where(m == gm, ci, NC), axis=1, keepdims=True)
        outs.append(base + node * NC + sel)
        m = jnp.where(ci == sel, neg, m)
    gidx_ref[...] = jnp.concatenate(outs, axis=1)


_knn = pl.pallas_call(
    _knn_body,
    grid=(B, N // RA),
    in_specs=[pl.BlockSpec((1, 4, N), lambda b, r: (b, 0, 0)),
              pl.BlockSpec((1, 4, RA), lambda b, r: (b, 0, r))],
    out_specs=[
        pl.BlockSpec((RA, N), lambda b, r: (b * (N // RA) + r, 0)),
        pl.BlockSpec((RA, KP1), lambda b, r: (b * (N // RA) + r, 0)),
    ],
    out_shape=[
        jax.ShapeDtypeStruct((NN, N), _F32),
        jax.ShapeDtypeStruct((NN, KP1), jnp.int32),
    ],
)


def _make_sc_gather(n_idx, width, chunk):
    per_w = n_idx // 32
    n_it = per_w // chunk
    assert per_w % chunk == 0 and chunk % 8 == 0 and per_w % 8 == 0
    mesh = plsc.VectorSubcoreMesh(core_axis_name="c", subcore_axis_name="s")

    @functools.partial(
        pl.kernel, mesh=mesh,
        compiler_params=pltpu.CompilerParams(use_tc_tiling_on_sc=False),
        out_type=jax.ShapeDtypeStruct((n_idx, width), _F32),
        scratch_types=[
            pltpu.VMEM((chunk,), jnp.int32),
            pltpu.VMEM((chunk, width), _F32),
            pltpu.SemaphoreType.DMA,
        ],
    )
    def gath(tab, idx, out, idx_v, rows_v, sem):
        wid = lax.axis_index("s") * 2 + lax.axis_index("c")
        base = wid * per_w

        def body(i, c):
            off = base + i * chunk
            pltpu.sync_copy(idx.at[pl.ds(off, chunk)], idx_v)
            pltpu.async_copy(tab.at[idx_v], rows_v, sem).wait()
            pltpu.sync_copy(rows_v, out.at[pl.ds(off, chunk)])
            return c

        lax.fori_loop(0, n_it, body, 0)

    return gath


_gather_cand = _make_sc_gather(NN * KP1, CH, 1088)
_gather_wt = _make_sc_gather(E, DIN, 2048)


def _select_body(cand_ref, gidx_ref, fr_ref, x_ref, er_ref, m_ref,
                 idx_ref, wt_ref):
    vals = cand_ref[...]
    g = gidx_ref[...]
    tcf = lax.rem(g, NC).astype(_F32)
    tcp = jnp.concatenate([tcf, jnp.zeros((RC, 32 - KP1), _F32)], axis=1)
    rep = _mm(tcp, er_ref[...])
    si = lax.broadcasted_iota(jnp.int32, (RC, KP1 * CH), 1)
    cidx = rep * CH + lax.rem(si, CH).astype(_F32)
    boff = (pl.program_id(0) * RC // N) * N
    bigc = _F32(N)
    neg = _F32(-jnp.inf)
    outs = []
    for t in range(KP1):
        gm = jnp.max(vals, axis=1, keepdims=True)
        sel = jnp.min(jnp.where(vals == gm, cidx, bigc), axis=1, keepdims=True)
        if t > 0:
            outs.append(sel.astype(jnp.int32) + boff)
        vals = jnp.where(cidx == sel, neg, vals)
    idx_ref[...] = jnp.concatenate(outs, axis=1)

    x = x_ref[...]
    fr = fr_ref[...]
    mats = m_ref[...]
    w = jnp.zeros((RC, DIN), _F32)
    for b in range(4):
        w = w + (_mm(fr, mats[b * 16:(b + 1) * 16, :])
                 * _mm(x, mats[64 + b * 16:64 + (b + 1) * 16, :]))
    wt_ref[...] = w


_select = pl.pallas_call(
    _select_body,
    grid=(NN // RC,),
    in_specs=[
        pl.BlockSpec((RC, KP1 * CH), lambda i: (i, 0)),
        pl.BlockSpec((RC, KP1), lambda i: (i, 0)),
        pl.BlockSpec((RC, DIN), lambda i: (i, 0)),
        pl.BlockSpec((RC, DIN), lambda i: (i, 0)),
        pl.BlockSpec((32, KP1 * CH), lambda i: (0, 0)),
        pl.BlockSpec((128, DIN), lambda i: (0, 0)),
    ],
    out_specs=[
        pl.BlockSpec((RC, KNB), lambda i: (i, 0)),
        pl.BlockSpec((RC, DIN), lambda i: (i, 0)),
    ],
    out_shape=[
        jax.ShapeDtypeStruct((NN, KNB), jnp.int32),
        jax.ShapeDtypeStruct((NN, DIN), _F32),
    ],
)


def _l1_body(wg_ref, fr_ref, x_ref, wb_ref, wa_ref, wsc_ref, m_ref,
             y1_ref, scp_ref, st_ref):
    wg = wg_ref[...]
    fr = fr_ref[...]
    x = x_ref[...]
    mats = m_ref[...]
    fie = jnp.broadcast_to(fr[:, None, :], (ND, KNB, DIN)).reshape(ND * KNB, DIN)
    fts = jnp.zeros((ND * KNB, DIN), _F32)
    for b in range(4):
        fts = fts + (_mm(fie, mats[b * 16:(b + 1) * 16, :])
                     * _mm(wg, mats[64 + b * 16:64 + (b + 1) * 16, :]))
    nt = _mm(x, wa_ref[...])
    nte = jnp.broadcast_to(nt[:, None, :], (ND, KNB, 64)).reshape(ND * KNB, 64)
    y1 = _mm(fts, wb_ref[...]) + nte
    scp = _mm(x, wsc_ref[...])
    y1_ref[...] = y1
    scp_ref[...] = scp
    blk = jnp.concatenate([
        jnp.sum(y1, axis=0, keepdims=True),
        jnp.sum(y1 * y1, axis=0, keepdims=True),
        jnp.sum(scp, axis=0, keepdims=True),
        jnp.sum(scp * scp, axis=0, keepdims=True),
        jnp.zeros((4, 64), _F32),
    ], axis=0)

    @pl.when(pl.program_id(0) == 0)
    def _():
        st_ref[...] = jnp.zeros_like(st_ref)

    st_ref[...] += blk


_l1 = pl.pallas_call(
    _l1_body,
    grid=(NN // ND,),
    in_specs=[
        pl.BlockSpec((ND * KNB, DIN), lambda i: (i, 0)),
        pl.BlockSpec((ND, DIN), lambda i: (i, 0)),
        pl.BlockSpec((ND, DIN), lambda i: (i, 0)),
        pl.BlockSpec((DIN, 64), lambda i: (0, 0)),
        pl.BlockSpec((DIN, 64), lambda i: (0, 0)),
        pl.BlockSpec((DIN, 64), lambda i: (0, 0)),
        pl.BlockSpec((128, DIN), lambda i: (0, 0)),
    ],
    out_specs=[
        pl.BlockSpec((ND * KNB, 64), lambda i: (i, 0)),
        pl.BlockSpec((ND, 64), lambda i: (i, 0)),
        pl.BlockSpec((8, 64), lambda i: (0, 0)),
    ],
    out_shape=[
        jax.ShapeDtypeStruct((E, 64), _F32),
        jax.ShapeDtypeStruct((NN, 64), _F32),
        jax.ShapeDtypeStruct((8, 64), _F32),
    ],
)


def _mlp_body(y_ref, p_ref, w_ref, o_ref, st_ref):
    prm = p_ref[...]
    a = jnp.maximum(y_ref[...] * prm[0:1, :] + prm[1:2, :], 0.0)
    y2 = _mm(a, w_ref[...])
    o_ref[...] = y2
    blk = jnp.concatenate([
        jnp.sum(y2, axis=0, keepdims=True),
        jnp.sum(y2 * y2, axis=0, keepdims=True),
        jnp.zeros((6, 64), _F32),
    ], axis=0)

    @pl.when(pl.program_id(0) == 0)
    def _():
        st_ref[...] = jnp.zeros_like(st_ref)

    st_ref[...] += blk


_mlp = pl.pallas_call(
    _mlp_body,
    grid=(E // RE,),
    in_specs=[
        pl.BlockSpec((RE, 64), lambda i: (i, 0)),
        pl.BlockSpec((8, 64), lambda i: (0, 0)),
        pl.BlockSpec((64, 64), lambda i: (0, 0)),
    ],
    out_specs=[
        pl.BlockSpec((RE, 64), lambda i: (i, 0)),
        pl.BlockSpec((8, 64), lambda i: (0, 0)),
    ],
    out_shape=[
        jax.ShapeDtypeStruct((E, 64), _F32),
        jax.ShapeDtypeStruct((8, 64), _F32),
    ],
)


def _out_body(y3_ref, scp_ref, p_ref, o_ref):
    prm = p_ref[...]
    a3 = jnp.maximum(y3_ref[...] * prm[0:1, :] + prm[1:2, :], 0.0)
    mean = jnp.sum(a3.reshape(ND, KNB, 64), axis=1) * _F32(1.0 / KNB)
    sc = scp_ref[...] * prm[2:3, :] + prm[3:4, :]
    o_ref[...] = jnp.maximum(sc + mean, 0.0)


_outk = pl.pallas_call(
    _out_body,
    grid=(NN // ND,),
    in_specs=[
        pl.BlockSpec((ND * KNB, 64), lambda i: (i, 0)),
        pl.BlockSpec((ND, 64), lambda i: (i, 0)),
        pl.BlockSpec((8, 64), lambda i: (0, 0)),
    ],
    out_specs=pl.BlockSpec((ND, 64), lambda i: (i, 0)),
    out_shape=jax.ShapeDtypeStruct((NN, 64), _F32),
)


def _bn_affine(s, ss, cnt, g, b):
    mu = s / cnt
    var = ss / cnt - mu * mu
    sc = g / jnp.sqrt(var + EPS)
    return sc, b - mu * sc


def kernel(points, features, frames, W1, g1, b1, W2, g2, b2, W3, g3, b3,
           Wsc, gsc, bsc):
    xT = jnp.transpose(features, (0, 2, 1)).reshape(NN, DIN)
    frf = frames.reshape(NN, DIN)

    score, gidx = _knn(points, points)
    cand = _gather_cand(score.reshape(NN * NC, CH), gidx.reshape(-1))
    idx16, wt = _select(cand.reshape(NN, KP1 * CH), gidx, frf, xT,
                        jnp.asarray(_EREP), jnp.asarray(_SELM))
    wg = _gather_wt(wt, idx16.reshape(-1))

    W1a, W1b = W1[:, :DIN], W1[:, DIN:]
    y1, scp, st1 = _l1(wg, frf, xT, W1b.T, (W1a - W1b).T, Wsc.T,
                       jnp.asarray(_L1M))

    s1, t1 = _bn_affine(st1[0], st1[1], _F32(E), g1, b1)
    ssc, tsc = _bn_affine(st1[2], st1[3], _F32(NN), gsc, bsc)
    prm1 = jnp.concatenate([s1[None], t1[None], jnp.zeros((6, 64), _F32)], 0)
    y2, st2 = _mlp(y1, prm1, W2.T)

    s2, t2 = _bn_affine(st2[0], st2[1], _F32(E), g2, b2)
    prm2 = jnp.concatenate([s2[None], t2[None], jnp.zeros((6, 64), _F32)], 0)
    y3, st3 = _mlp(y2, prm2, W3.T)

    s3, t3 = _bn_affine(st3[0], st3[1], _F32(E), g3, b3)
    prm3 = jnp.concatenate([s3[None], t3[None], ssc[None], tsc[None],
                            jnp.zeros((4, 64), _F32)], 0)
    out = _outk(y3, scp, prm3)
    return out.reshape(B, N, 64).transpose(0, 2, 1)
